# baseline (speedup 1.0000x reference)
import functools
import math

import jax
import jax.numpy as jnp
from jax import lax
from jax.experimental import pallas as pl
from jax.experimental.pallas import tpu as pltpu
from jax.experimental.pallas import tpu_sc as plsc

_ALPHA = 0.6
_RATIO = 0.5
_FILL = -3.0e38

_NC = 2
_NS = 16
_NW = _NC * _NS
_L = 16
_BLK = 128


def _make_edge_agg(n, quart, nblk):
    ept = nblk * _BLK
    nacc = n + _L
    rpt = nacc // _NS
    mesh = plsc.VectorSubcoreMesh(core_axis_name="c", subcore_axis_name="s")

    @functools.partial(
        pl.kernel,
        mesh=mesh,
        compiler_params=pltpu.CompilerParams(
            needs_layout_passes=False, use_tc_tiling_on_sc=False),
        out_type=[
            jax.ShapeDtypeStruct((2 * _NC, nacc, quart), jnp.float32),
            jax.ShapeDtypeStruct((_NW, n), jnp.float32),
        ],
        scratch_types=[
            pltpu.VMEM((ept,), jnp.int32),
            pltpu.VMEM((nblk, _BLK), jnp.int32),
            pltpu.VMEM((_BLK, quart), jnp.float32),
            pltpu.VMEM((_BLK, quart), jnp.float32),
            pltpu.VMEM((_BLK, quart), jnp.float32),
            pltpu.VMEM((nacc,), jnp.float32),
            pltpu.VMEM_SHARED((nacc, quart), jnp.float32),
            pltpu.SemaphoreType.DMA,
            pltpu.SemaphoreType.DMA,
            pltpu.SemaphoreType.DMA,
            pltpu.SemaphoreType.DMA,
        ],
    )
    def edge_agg(xt_hbm, srcf_hbm, dstb_hbm, out_h, out_d,
                 src_v, dstb_v, rows_a, rows_b, zb_v, acc_d, acc_sh,
                 sem_a, sem_b, sem_sa, sem_sb):
        c = lax.axis_index("c")
        s = lax.axis_index("s")
        wid = s * _NC + c
        pltpu.sync_copy(srcf_hbm.at[pl.ds(s * ept, ept)], src_v)
        pltpu.sync_copy(dstb_hbm.at[pl.ds(s * nblk, nblk)], dstb_v)

        zeros = jnp.zeros((_L,), jnp.float32)

        def zrow(i, carry):
            for l in range(quart // _L):
                zb_v[i, pl.ds(l * _L, _L)] = zeros
            return carry

        lax.fori_loop(0, _BLK, zrow, 0)

        def zdeg(i, carry):
            acc_d[pl.ds(i * _L, _L)] = zeros
            return carry

        lax.fori_loop(0, nacc // _L, zdeg, 0)

        rbase = s * rpt
        nfull = rpt // _BLK
        rem = rpt - nfull * _BLK

        for p in range(2):
            off = 2 * c * n if p == 0 else n

            def pbody(i, carry, _off=off):
                src_v[pl.ds(i * _L, _L)] = src_v[pl.ds(i * _L, _L)] + _off
                return carry

            lax.fori_loop(0, ept // _L, pbody, 0)

            for q in range(nfull):
                pltpu.sync_copy(zb_v,
                                acc_sh.at[pl.ds(rbase + q * _BLK, _BLK)])
            if rem:
                pltpu.sync_copy(zb_v.at[pl.ds(0, rem)],
                                acc_sh.at[pl.ds(rbase + nfull * _BLK, rem)])
            plsc.subcore_barrier()

            def gather(blk, buf, sem):
                idx = src_v.at[pl.ds(blk * _BLK, _BLK)]
                pltpu.async_copy(xt_hbm.at[idx], buf, sem)

            def gwait(buf, sem):
                pltpu.make_async_copy(xt_hbm.at[pl.ds(0, _BLK)], buf,
                                      sem).wait()

            def scat(blk, buf, sem):
                pltpu.async_copy(buf, acc_sh.at[dstb_v.at[blk]], sem,
                                 add=True)

            def swait(buf, sem):
                pltpu.make_async_copy(buf, acc_sh.at[dstb_v.at[0]],
                                      sem).wait()

            gather(0, rows_a, sem_a)
            gather(1, rows_b, sem_b)

            def ebody(i, carry):
                gwait(rows_a, sem_a)
                scat(2 * i, rows_a, sem_sa)
                gwait(rows_b, sem_b)
                scat(2 * i + 1, rows_b, sem_sb)
                swait(rows_a, sem_sa)
                gather(lax.rem(2 * i + 2, nblk), rows_a, sem_a)
                swait(rows_b, sem_sb)
                gather(lax.rem(2 * i + 3, nblk), rows_b, sem_b)
                return carry

            lax.fori_loop(0, nblk // 2, ebody, 0)
            gwait(rows_a, sem_a)
            gwait(rows_b, sem_b)
            plsc.subcore_barrier()
            pltpu.sync_copy(acc_sh.at[pl.ds(rbase, rpt)],
                            out_h.at[2 * c + p, pl.ds(rbase, rpt)])
            plsc.subcore_barrier()

        ones = jnp.full((_L,), 1.0, jnp.float32)

        def dbody(j, carry):
            for l in range(_BLK // _L):
                di = dstb_v[j, pl.ds(l * _L, _L)]
                plsc.addupdate_scatter(acc_d, [di], ones)
            return carry

        lax.fori_loop(0, nblk, dbody, 0)
        pltpu.sync_copy(acc_d.at[pl.ds(0, n)], out_d.at[wid])

    return edge_agg


def _score_body(x_ref, a0_ref, a1_ref, a2_ref, a3_ref, pdt_ref, w2_ref,
                wn_ref, bs_ref, bf_ref, o_ref):
    dg = 0.5 * jnp.sum(pdt_ref[...], axis=1, keepdims=True)
    dgc = jnp.clip(dg, 1.0, None)
    h = jnp.concatenate([a0_ref[...], a1_ref[...], a2_ref[...],
                         a3_ref[...]], axis=1) / dgc
    hnwn = jnp.dot(h, wn_ref[...])
    ysf = jnp.dot(x_ref[...], w2_ref[...])
    score_s = ysf[:, 0:1] + hnwn + bs_ref[0, 0]
    score_f = ysf[:, 1:2] + bf_ref[0, 0]
    o_ref[...] = score_s * _ALPHA + score_f * (1.0 - _ALPHA)


def _make_score(n, d, nw, rows):
    quart = d // 4
    grid = n // rows
    return pl.pallas_call(
        _score_body,
        grid=(grid,),
        in_specs=[
            pl.BlockSpec((rows, d), lambda i: (i, 0)),
            pl.BlockSpec((rows, quart), lambda i: (i, 0)),
            pl.BlockSpec((rows, quart), lambda i: (i, 0)),
            pl.BlockSpec((rows, quart), lambda i: (i, 0)),
            pl.BlockSpec((rows, quart), lambda i: (i, 0)),
            pl.BlockSpec((rows, nw), lambda i: (i, 0)),
            pl.BlockSpec((d, 2), lambda i: (0, 0)),
            pl.BlockSpec((d, 1), lambda i: (0, 0)),
            pl.BlockSpec((1, 1), lambda i: (0, 0)),
            pl.BlockSpec((1, 1), lambda i: (0, 0)),
        ],
        out_specs=pl.BlockSpec((rows, 1), lambda i: (i, 0)),
        out_shape=jax.ShapeDtypeStruct((n, 1), jnp.float32),
    )


def _bitonic_desc(key, idx):
    b, m = key.shape
    lane = lax.broadcasted_iota(jnp.int32, (b, m), 1)
    sz = 2
    while sz <= m:
        st = sz // 2
        while st >= 1:
            lower = (lane & st) == 0
            pk = jnp.where(lower, jnp.roll(key, -st, axis=1),
                           jnp.roll(key, st, axis=1))
            pi = jnp.where(lower, jnp.roll(idx, -st, axis=1),
                           jnp.roll(idx, st, axis=1))
            desc = (lane & sz) == 0
            better = (key > pk) | ((key == pk) & (idx < pi))
            take_self = better == (lower == desc)
            key = jnp.where(take_self, key, pk)
            idx = jnp.where(take_self, idx, pi)
            st //= 2
        sz *= 2
    return key, idx


def _make_sort(b, m, n_per, kout):
    def body(sc_ref, o_ref):
        row = lax.broadcasted_iota(jnp.int32, (b, m), 0)
        col = lax.broadcasted_iota(jnp.int32, (b, m), 1)
        idx = row * n_per + col
        _, idx = _bitonic_desc(sc_ref[...], idx)
        o_ref[...] = idx[:, :kout]

    return pl.pallas_call(
        body,
        out_shape=jax.ShapeDtypeStruct((b, kout), jnp.int32),
    )


def kernel(x, edge_index, batch_num_nodes, W_self, W_neigh, b_s, W_f, b_f):
    n, d = x.shape
    b = batch_num_nodes.shape[0]
    n_per = n // b
    m = 1 << (n_per - 1).bit_length()
    k_static = int(math.ceil(_RATIO * n_per))
    kout = ((k_static + 127) // 128) * 128

    e = edge_index.shape[1]
    nblk = (e + _NS * _BLK - 1) // (_NS * _BLK)
    nblk += nblk % 2
    ept = nblk * _BLK
    e_pad = ept * _NS
    pad = e_pad - e
    src_p = jnp.concatenate([edge_index[0], jnp.zeros((pad,), jnp.int32)])
    dst_p = jnp.concatenate([edge_index[1], jnp.full((pad,), n, jnp.int32)])
    quart = d // 4
    xt = x.reshape(n, 4, quart).transpose(1, 0, 2).reshape(4 * n, quart)
    agg, pd = _make_edge_agg(n, quart, nblk)(
        xt, src_p, dst_p.reshape(_NS * nblk, _BLK))

    aq = [agg[q, :n, :] for q in range(4)]
    w2 = jnp.concatenate([W_self, W_f], axis=1)
    score = _make_score(n, d, _NW, 2000)(
        x, aq[0], aq[1], aq[2], aq[3], pd.T, w2, W_neigh,
        b_s.reshape(1, 1), b_f.reshape(1, 1))

    sc2 = jnp.pad(score[:, 0].reshape(b, n_per), ((0, 0), (0, m - n_per)),
                  constant_values=_FILL)
    idx_sorted = _make_sort(b, m, n_per, kout)(sc2)

    perm = idx_sorted[:, :k_static].reshape(-1)
    k = jnp.ceil(_RATIO * batch_num_nodes.astype(x.dtype)).astype(jnp.int32)
    return (x, perm, k)


# speedup vs baseline: 1.0950x; 1.0950x over previous
import functools
import math

import jax
import jax.numpy as jnp
from jax import lax
from jax.experimental import pallas as pl
from jax.experimental.pallas import tpu as pltpu
from jax.experimental.pallas import tpu_sc as plsc

_ALPHA = 0.6
_RATIO = 0.5
_FILL = -3.0e38

_NC = 2
_NS = 16
_NW = _NC * _NS
_L = 16
_BLK = 128


def _make_edge_agg(n, quart, nblk):
    ept = nblk * _BLK
    nacc = n + _L
    rpt = nacc // _NS
    mesh = plsc.VectorSubcoreMesh(core_axis_name="c", subcore_axis_name="s")

    @functools.partial(
        pl.kernel,
        mesh=mesh,
        compiler_params=pltpu.CompilerParams(
            needs_layout_passes=False, use_tc_tiling_on_sc=False),
        out_type=[
            jax.ShapeDtypeStruct((2 * _NC, nacc, quart), jnp.float32),
            jax.ShapeDtypeStruct((_NW, n), jnp.float32),
        ],
        scratch_types=[
            pltpu.VMEM((ept,), jnp.int32),
            pltpu.VMEM((nblk, _BLK), jnp.int32),
            pltpu.VMEM((_BLK, quart), jnp.float32),
            pltpu.VMEM((_BLK, quart), jnp.float32),
            pltpu.VMEM((nacc,), jnp.float32),
            pltpu.VMEM_SHARED((nacc, quart), jnp.float32),
            pltpu.SemaphoreType.DMA,
        ],
    )
    def edge_agg(xt_hbm, srcf_hbm, dstb_hbm, out_h, out_d,
                 src_v, dstb_v, rows_a, zb_v, acc_d, acc_sh, sem_a):
        c = lax.axis_index("c")
        s = lax.axis_index("s")
        wid = s * _NC + c
        pltpu.sync_copy(srcf_hbm.at[pl.ds(s * ept, ept)], src_v)
        pltpu.sync_copy(dstb_hbm.at[pl.ds(s * nblk, nblk)], dstb_v)

        zeros = jnp.zeros((_L,), jnp.float32)

        def zrow(i, carry):
            for l in range(quart // _L):
                zb_v[i, pl.ds(l * _L, _L)] = zeros
            return carry

        lax.fori_loop(0, _BLK, zrow, 0)

        def zdeg(i, carry):
            acc_d[pl.ds(i * _L, _L)] = zeros
            return carry

        lax.fori_loop(0, nacc // _L, zdeg, 0)

        rbase = s * rpt
        nfull = rpt // _BLK
        rem = rpt - nfull * _BLK

        for p in range(2):
            off = 2 * c * n if p == 0 else n

            def pbody(i, carry, _off=off):
                src_v[pl.ds(i * _L, _L)] = src_v[pl.ds(i * _L, _L)] + _off
                return carry

            lax.fori_loop(0, ept // _L, pbody, 0)

            for q in range(nfull):
                pltpu.sync_copy(zb_v,
                                acc_sh.at[pl.ds(rbase + q * _BLK, _BLK)])
            if rem:
                pltpu.sync_copy(zb_v.at[pl.ds(0, rem)],
                                acc_sh.at[pl.ds(rbase + nfull * _BLK, rem)])
            plsc.subcore_barrier()

            def ebody(j, carry):
                idx = src_v.at[pl.ds(j * _BLK, _BLK)]
                pltpu.async_copy(xt_hbm.at[idx], rows_a, sem_a).wait()
                pltpu.sync_copy(rows_a, acc_sh.at[dstb_v.at[j]], add=True)
                return carry

            lax.fori_loop(0, nblk, ebody, 0)
            plsc.subcore_barrier()
            pltpu.sync_copy(acc_sh.at[pl.ds(rbase, rpt)],
                            out_h.at[2 * c + p, pl.ds(rbase, rpt)])
            plsc.subcore_barrier()

        ones = jnp.full((_L,), 1.0, jnp.float32)

        def dbody(j, carry):
            for l in range(_BLK // _L):
                di = dstb_v[j, pl.ds(l * _L, _L)]
                plsc.addupdate_scatter(acc_d, [di], ones)
            return carry

        lax.fori_loop(0, nblk, dbody, 0)
        pltpu.sync_copy(acc_d.at[pl.ds(0, n)], out_d.at[wid])

    return edge_agg


def _score_body(x_ref, a0_ref, a1_ref, a2_ref, a3_ref, pdt_ref, w2_ref,
                wn_ref, bs_ref, bf_ref, o_ref):
    dg = 0.5 * jnp.sum(pdt_ref[...], axis=1, keepdims=True)
    dgc = jnp.clip(dg, 1.0, None)
    h = jnp.concatenate([a0_ref[...], a1_ref[...], a2_ref[...],
                         a3_ref[...]], axis=1) / dgc
    hnwn = jnp.dot(h, wn_ref[...])
    ysf = jnp.dot(x_ref[...], w2_ref[...])
    score_s = ysf[:, 0:1] + hnwn + bs_ref[0, 0]
    score_f = ysf[:, 1:2] + bf_ref[0, 0]
    o_ref[...] = score_s * _ALPHA + score_f * (1.0 - _ALPHA)


def _make_score(n, d, nw, rows):
    quart = d // 4
    grid = n // rows
    return pl.pallas_call(
        _score_body,
        grid=(grid,),
        in_specs=[
            pl.BlockSpec((rows, d), lambda i: (i, 0)),
            pl.BlockSpec((rows, quart), lambda i: (i, 0)),
            pl.BlockSpec((rows, quart), lambda i: (i, 0)),
            pl.BlockSpec((rows, quart), lambda i: (i, 0)),
            pl.BlockSpec((rows, quart), lambda i: (i, 0)),
            pl.BlockSpec((rows, nw), lambda i: (i, 0)),
            pl.BlockSpec((d, 2), lambda i: (0, 0)),
            pl.BlockSpec((d, 1), lambda i: (0, 0)),
            pl.BlockSpec((1, 1), lambda i: (0, 0)),
            pl.BlockSpec((1, 1), lambda i: (0, 0)),
        ],
        out_specs=pl.BlockSpec((rows, 1), lambda i: (i, 0)),
        out_shape=jax.ShapeDtypeStruct((n, 1), jnp.float32),
    )


def _bitonic_desc(key, idx):
    b, m = key.shape
    lane = lax.broadcasted_iota(jnp.int32, (b, m), 1)
    sz = 2
    while sz <= m:
        st = sz // 2
        while st >= 1:
            lower = (lane & st) == 0
            pk = jnp.where(lower, jnp.roll(key, -st, axis=1),
                           jnp.roll(key, st, axis=1))
            pi = jnp.where(lower, jnp.roll(idx, -st, axis=1),
                           jnp.roll(idx, st, axis=1))
            desc = (lane & sz) == 0
            better = (key > pk) | ((key == pk) & (idx < pi))
            take_self = better == (lower == desc)
            key = jnp.where(take_self, key, pk)
            idx = jnp.where(take_self, idx, pi)
            st //= 2
        sz *= 2
    return key, idx


def _make_sort(b, m, n_per, kout):
    def body(sc_ref, o_ref):
        row = lax.broadcasted_iota(jnp.int32, (b, m), 0)
        col = lax.broadcasted_iota(jnp.int32, (b, m), 1)
        idx = row * n_per + col
        _, idx = _bitonic_desc(sc_ref[...], idx)
        o_ref[...] = idx[:, :kout]

    return pl.pallas_call(
        body,
        out_shape=jax.ShapeDtypeStruct((b, kout), jnp.int32),
    )


def kernel(x, edge_index, batch_num_nodes, W_self, W_neigh, b_s, W_f, b_f):
    n, d = x.shape
    b = batch_num_nodes.shape[0]
    n_per = n // b
    m = 1 << (n_per - 1).bit_length()
    k_static = int(math.ceil(_RATIO * n_per))
    kout = ((k_static + 127) // 128) * 128

    e = edge_index.shape[1]
    nblk = (e + _NS * _BLK - 1) // (_NS * _BLK)
    ept = nblk * _BLK
    e_pad = ept * _NS
    pad = e_pad - e
    src_p = jnp.concatenate([edge_index[0], jnp.zeros((pad,), jnp.int32)])
    dst_p = jnp.concatenate([edge_index[1], jnp.full((pad,), n, jnp.int32)])
    quart = d // 4
    xt = x.reshape(n, 4, quart).transpose(1, 0, 2).reshape(4 * n, quart)
    agg, pd = _make_edge_agg(n, quart, nblk)(
        xt, src_p, dst_p.reshape(_NS * nblk, _BLK))

    aq = [agg[q, :n, :] for q in range(4)]
    w2 = jnp.concatenate([W_self, W_f], axis=1)
    score = _make_score(n, d, _NW, 2000)(
        x, aq[0], aq[1], aq[2], aq[3], pd.T, w2, W_neigh,
        b_s.reshape(1, 1), b_f.reshape(1, 1))

    sc2 = jnp.pad(score[:, 0].reshape(b, n_per), ((0, 0), (0, m - n_per)),
                  constant_values=_FILL)
    idx_sorted = _make_sort(b, m, n_per, kout)(sc2)

    perm = idx_sorted[:, :k_static].reshape(-1)
    k = jnp.ceil(_RATIO * batch_num_nodes.astype(x.dtype)).astype(jnp.int32)
    return (x, perm, k)
